# baseline (speedup 1.0000x reference)
import functools

import jax
import jax.numpy as jnp
from jax import lax
from jax.experimental import pallas as pl
from jax.experimental.pallas import tpu as pltpu
from jax.experimental.pallas import tpu_sc as plsc


def _proj_body(emb_ref, wt_ref, b_ref, out_ref):
    out_ref[...] = (
        jnp.dot(emb_ref[...], wt_ref[...], preferred_element_type=jnp.float32)
        + b_ref[...]
    )


def _project(emb4, w4, b4):
    V4, L = emb4.shape
    M = w4.shape[1]
    R = 25000
    grid = V4 // R
    return pl.pallas_call(
        _proj_body,
        grid=(grid,),
        in_specs=[
            pl.BlockSpec((R, L), lambda i: (i, 0)),
            pl.BlockSpec((L, M), lambda i: (0, 0)),
            pl.BlockSpec((1, M), lambda i: (0, 0)),
        ],
        out_specs=pl.BlockSpec((R, M), lambda i: (i, 0)),
        out_shape=jax.ShapeDtypeStruct((V4, M), jnp.float32),
    )(emb4, w4, b4)


@functools.lru_cache(maxsize=None)
def _make_gather(V, N, B):
    info = plsc.get_sparse_core_info()
    NC, NS = info.num_cores, info.num_subcores
    NW = NC * NS
    assert B % NW == 0
    b_per_w = B // NW
    mesh = plsc.VectorSubcoreMesh(core_axis_name="c", subcore_axis_name="s")

    IW = 128
    assert b_per_w % IW == 0
    rows_per_w = b_per_w // IW
    G = 10
    NB = 2
    assert rows_per_w % (G * NB) == 0
    n_steps = rows_per_w // (G * NB)

    NO = 4

    @functools.partial(
        pl.kernel,
        mesh=mesh,
        out_type=jax.ShapeDtypeStruct((B, NO), jnp.float32),
        scratch_types=[
            pltpu.VMEM((rows_per_w, IW), jnp.int32),
            pltpu.VMEM((G * IW, N), jnp.float32),
            pltpu.VMEM((G * IW, N), jnp.float32),
            pltpu.SemaphoreType.DMA,
            pltpu.SemaphoreType.DMA,
        ],
        compiler_params=pltpu.CompilerParams(use_tc_tiling_on_sc=False),
    )
    def gather(proj_hbm, idx_hbm, out_hbm, idx_v, rows0, rows1, sem0, sem1):
        wid = lax.axis_index("s") * NC + lax.axis_index("c")
        base = wid * b_per_w
        pltpu.sync_copy(idx_hbm.at[pl.ds(wid * rows_per_w, rows_per_w)], idx_v)
        bufs = (rows0, rows1)
        sems = (sem0, sem1)

        def fire(g2, b):
            g = g2 * NB + b
            return [
                pltpu.async_copy(
                    proj_hbm.at[idx_v.at[g * G + r]],
                    bufs[b].at[pl.ds(r * IW, IW)],
                    sems[b],
                )
                for r in range(G)
            ]

        def drain(g2, b, copies):
            g = g2 * NB + b
            for c in copies:
                c.wait()
            pltpu.sync_copy(
                bufs[b].at[:, pl.ds(0, NO)],
                out_hbm.at[pl.ds(base + g * G * IW, G * IW)],
            )

        def step(g2, carry):
            copies = [fire(g2, b) for b in range(NB)]
            for b in range(NB):
                drain(g2, b, copies[b])
            return carry

        lax.fori_loop(0, n_steps, step, 0)

    return gather


def kernel(text, emb_table, fc_w, fc_b):
    Bt, S = text.shape
    V, D = emb_table.shape
    N = fc_w.shape[0]
    NP = 8
    K = 4
    wt = jnp.zeros((D, NP), jnp.float32).at[:, :N].set(fc_w.T)
    w4 = jnp.zeros((K, D, K, NP), jnp.float32)
    w4 = w4.at[jnp.arange(K), :, jnp.arange(K), :].set(wt).reshape(K * D, K * NP)
    b4 = jnp.tile(jnp.zeros((NP,), jnp.float32).at[:N].set(fc_b), K).reshape(1, K * NP)
    emb4 = emb_table.reshape(V // K, K * D)
    proj = _project(emb4, w4, b4).reshape(V, NP)
    idx = text.astype(jnp.int32).reshape(-1, 128)
    out = _make_gather(V, NP, idx.size)(proj, idx)
    return out.reshape(Bt, S, N)


# speedup vs baseline: 2.8246x; 2.8246x over previous
import functools

import jax
import jax.numpy as jnp
from jax import lax
from jax.experimental import pallas as pl
from jax.experimental.pallas import tpu as pltpu
from jax.experimental.pallas import tpu_sc as plsc


def _proj_t_body(x_ref, w_ref, b_ref, out_ref):
    out_ref[...] = (
        lax.dot_general(
            x_ref[...], w_ref[...], (((0,), (0,)), ((), ())),
            preferred_element_type=jnp.float32,
        )
        + b_ref[...]
    )


def _project_t(x512, w512, b128):
    DK, Q = x512.shape
    CB = 4096
    grid = Q // CB
    return pl.pallas_call(
        _proj_t_body,
        grid=(grid,),
        in_specs=[
            pl.BlockSpec((DK, CB), lambda i: (0, i)),
            pl.BlockSpec((DK, 128), lambda i: (0, 0)),
            pl.BlockSpec((1, 128), lambda i: (0, 0)),
        ],
        out_specs=pl.BlockSpec((CB, 128), lambda i: (i, 0)),
        out_shape=jax.ShapeDtypeStruct((Q, 128), jnp.float32),
    )(x512, w512, b128)


@functools.lru_cache(maxsize=None)
def _make_gather(V, N, B):
    info = plsc.get_sparse_core_info()
    NC, NS = info.num_cores, info.num_subcores
    NW = NC * NS
    assert B % NW == 0
    b_per_w = B // NW
    mesh = plsc.VectorSubcoreMesh(core_axis_name="c", subcore_axis_name="s")

    IW = 128
    assert b_per_w % IW == 0
    rows_per_w = b_per_w // IW
    G = 10
    NB = 2
    assert rows_per_w % (G * NB) == 0
    n_steps = rows_per_w // (G * NB)

    @functools.partial(
        pl.kernel,
        mesh=mesh,
        out_type=jax.ShapeDtypeStruct((B, N), jnp.float32),
        scratch_types=[
            pltpu.VMEM((rows_per_w, IW), jnp.int32),
            pltpu.VMEM((G * IW, N), jnp.float32),
            pltpu.VMEM((G * IW, N), jnp.float32),
            pltpu.SemaphoreType.DMA,
            pltpu.SemaphoreType.DMA,
        ],
        compiler_params=pltpu.CompilerParams(use_tc_tiling_on_sc=False),
    )
    def gather(proj_hbm, idx_hbm, out_hbm, idx_v, rows0, rows1, sem0, sem1):
        wid = lax.axis_index("s") * NC + lax.axis_index("c")
        base = wid * b_per_w
        pltpu.sync_copy(idx_hbm.at[pl.ds(wid * rows_per_w, rows_per_w)], idx_v)
        bufs = (rows0, rows1)
        sems = (sem0, sem1)

        def fire(g2, b):
            g = g2 * NB + b
            return [
                pltpu.async_copy(
                    proj_hbm.at[idx_v.at[g * G + r]],
                    bufs[b].at[pl.ds(r * IW, IW)],
                    sems[b],
                )
                for r in range(G)
            ]

        def drain(g2, b, copies):
            g = g2 * NB + b
            for c in copies:
                c.wait()
            pltpu.sync_copy(bufs[b], out_hbm.at[pl.ds(base + g * G * IW, G * IW)])

        def step(g2, carry):
            copies = [fire(g2, b) for b in range(NB)]
            for b in range(NB):
                drain(g2, b, copies[b])
            return carry

        lax.fori_loop(0, n_steps, step, 0)

    return gather


def kernel(text, emb_table, fc_w, fc_b):
    Bt, S = text.shape
    V, D = emb_table.shape
    N = fc_w.shape[0]
    NP = 8
    VP = 1048576

    PACK = 128 // NP
    VQ = VP // PACK

    wt = jnp.zeros((D, NP), jnp.float32).at[:, :N].set(fc_w.T)
    ar = jnp.arange(PACK)
    w512 = (
        jnp.zeros((D, PACK, PACK, NP), jnp.float32)
        .at[:, ar, ar, :]
        .set(jnp.broadcast_to(wt[:, None, :], (D, PACK, NP)))
        .reshape(D * PACK, PACK * NP)
    )
    b128 = jnp.tile(jnp.zeros((NP,), jnp.float32).at[:N].set(fc_b), PACK).reshape(1, 128)

    embt = jnp.pad(emb_table.T, ((0, 0), (0, VP - V)))
    x512 = embt.reshape(D * PACK, VQ)
    proj = _project_t(x512, w512, b128).reshape(VP, NP)
    tix = text.astype(jnp.int32)
    tix = ((tix % VQ) * PACK) + (tix // VQ)
    idx = tix.reshape(-1, 128)
    out = _make_gather(VP, NP, idx.size)(proj, idx)
    return out[:, :N].reshape(Bt, S, N)


# speedup vs baseline: 3.8180x; 1.3517x over previous
import functools

import jax
import jax.numpy as jnp
from jax import lax
from jax.experimental import pallas as pl
from jax.experimental.pallas import tpu as pltpu
from jax.experimental.pallas import tpu_sc as plsc


def _make_proj_body(PACK):
    def body(*refs):
        x_refs = refs[:PACK]
        w_ref, b_ref, out_ref = refs[PACK:]
        x = jnp.concatenate([r[...] for r in x_refs], axis=0)
        out_ref[...] = (
            lax.dot_general(
                x, w_ref[...], (((0,), (0,)), ((), ())),
                preferred_element_type=jnp.float32,
            )
            + b_ref[...]
        )

    return body


def _project_t(embt, w512, b128, PACK):
    D, VP = embt.shape
    VQ = VP // PACK
    CB = 4096
    assert VQ % CB == 0
    grid = VQ // CB
    nb = VQ // CB

    def mk_index_map(p):
        return lambda i: (0, p * nb + i)

    return pl.pallas_call(
        _make_proj_body(PACK),
        grid=(grid,),
        in_specs=[
            pl.BlockSpec((D, CB), mk_index_map(p)) for p in range(PACK)
        ]
        + [
            pl.BlockSpec((D * PACK, 128), lambda i: (0, 0)),
            pl.BlockSpec((1, 128), lambda i: (0, 0)),
        ],
        out_specs=pl.BlockSpec((CB, 128), lambda i: (i, 0)),
        out_shape=jax.ShapeDtypeStruct((VQ, 128), jnp.float32),
    )(*([embt] * PACK), w512, b128)


@functools.lru_cache(maxsize=None)
def _make_gather(V, N, B):
    info = plsc.get_sparse_core_info()
    NC, NS = info.num_cores, info.num_subcores
    NW = NC * NS
    assert B % NW == 0
    b_per_w = B // NW
    mesh = plsc.VectorSubcoreMesh(core_axis_name="c", subcore_axis_name="s")

    IW = 128
    assert b_per_w % IW == 0
    rows_per_w = b_per_w // IW
    G = 10
    NB = 2
    assert rows_per_w % (G * NB) == 0
    n_steps = rows_per_w // (G * NB)

    @functools.partial(
        pl.kernel,
        mesh=mesh,
        out_type=jax.ShapeDtypeStruct((B, N), jnp.float32),
        scratch_types=[
            pltpu.VMEM((rows_per_w, IW), jnp.int32),
            pltpu.VMEM((G * IW, N), jnp.float32),
            pltpu.VMEM((G * IW, N), jnp.float32),
            pltpu.SemaphoreType.DMA,
            pltpu.SemaphoreType.DMA,
        ],
        compiler_params=pltpu.CompilerParams(use_tc_tiling_on_sc=False),
    )
    def gather(proj_hbm, idx_hbm, out_hbm, idx_v, rows0, rows1, sem0, sem1):
        wid = lax.axis_index("s") * NC + lax.axis_index("c")
        base = wid * b_per_w
        pltpu.sync_copy(idx_hbm.at[pl.ds(wid * rows_per_w, rows_per_w)], idx_v)
        bufs = (rows0, rows1)
        sems = (sem0, sem1)

        def fire(g2, b):
            g = g2 * NB + b
            return [
                pltpu.async_copy(
                    proj_hbm.at[idx_v.at[g * G + r]],
                    bufs[b].at[pl.ds(r * IW, IW)],
                    sems[b],
                )
                for r in range(G)
            ]

        def drain(g2, b, copies):
            g = g2 * NB + b
            for c in copies:
                c.wait()
            pltpu.sync_copy(bufs[b], out_hbm.at[pl.ds(base + g * G * IW, G * IW)])

        def step(g2, carry):
            copies = [fire(g2, b) for b in range(NB)]
            for b in range(NB):
                drain(g2, b, copies[b])
            return carry

        lax.fori_loop(0, n_steps, step, 0)

    return gather


def kernel(text, emb_table, fc_w, fc_b):
    Bt, S = text.shape
    V, D = emb_table.shape
    N = fc_w.shape[0]
    NP = 8
    VP = 1048576

    PACK = 128 // NP
    VQ = VP // PACK

    wt = jnp.zeros((D, NP), jnp.float32).at[:, :N].set(fc_w.T)
    ar = jnp.arange(PACK)
    w512 = (
        jnp.zeros((D, PACK, PACK, NP), jnp.float32)
        .at[:, ar, ar, :]
        .set(jnp.broadcast_to(wt[:, None, :], (D, PACK, NP)))
        .reshape(D * PACK, PACK * NP)
    )
    b128 = jnp.tile(jnp.zeros((NP,), jnp.float32).at[:N].set(fc_b), PACK).reshape(1, 128)

    embt = jnp.pad(emb_table.T, ((0, 0), (0, VP - V)))
    proj = _project_t(embt, w512, b128, PACK).reshape(VP, NP)
    tix = text.astype(jnp.int32)
    tix = ((tix % VQ) * PACK) + (tix // VQ)
    idx = tix.reshape(-1, 128)
    out = _make_gather(VP, NP, idx.size)(proj, idx)
    return out[:, :N].reshape(Bt, S, N)


# speedup vs baseline: 3.8234x; 1.0014x over previous
import functools

import jax
import jax.numpy as jnp
from jax import lax
from jax.experimental import pallas as pl
from jax.experimental.pallas import tpu as pltpu
from jax.experimental.pallas import tpu_sc as plsc


def _make_proj_body(PACK):
    def body(*refs):
        x_refs = refs[:PACK]
        w_ref, b_ref, out_ref = refs[PACK:]
        x = jnp.concatenate([r[...] for r in x_refs], axis=0)
        out_ref[...] = (
            lax.dot_general(
                x, w_ref[...], (((0,), (0,)), ((), ())),
                preferred_element_type=jnp.float32,
            )
            + b_ref[...]
        )

    return body


def _project_t(embt, w512, b128, PACK):
    D, VP = embt.shape
    VQ = VP // PACK
    CB = 4096
    assert VQ % CB == 0
    grid = VQ // CB
    nb = VQ // CB

    def mk_index_map(p):
        return lambda i: (0, p * nb + i)

    return pl.pallas_call(
        _make_proj_body(PACK),
        grid=(grid,),
        in_specs=[
            pl.BlockSpec((D, CB), mk_index_map(p)) for p in range(PACK)
        ]
        + [
            pl.BlockSpec((D * PACK, 128), lambda i: (0, 0)),
            pl.BlockSpec((1, 128), lambda i: (0, 0)),
        ],
        out_specs=pl.BlockSpec((CB, 128), lambda i: (i, 0)),
        out_shape=jax.ShapeDtypeStruct((VQ, 128), jnp.float32),
    )(*([embt] * PACK), w512, b128)


@functools.lru_cache(maxsize=None)
def _make_gather(V, N, B):
    info = plsc.get_sparse_core_info()
    NC, NS = info.num_cores, info.num_subcores
    NW = NC * NS
    assert B % NW == 0
    b_per_w = B // NW
    mesh = plsc.VectorSubcoreMesh(core_axis_name="c", subcore_axis_name="s")

    IW = 128
    assert b_per_w % IW == 0
    rows_per_w = b_per_w // IW
    G = 10
    NB = 2
    assert rows_per_w % (G * NB) == 0
    n_steps = rows_per_w // (G * NB)

    @functools.partial(
        pl.kernel,
        mesh=mesh,
        out_type=jax.ShapeDtypeStruct((B, N), jnp.float32),
        scratch_types=[
            pltpu.VMEM((rows_per_w, IW), jnp.int32),
            pltpu.VMEM((G * IW, N), jnp.float32),
            pltpu.VMEM((G * IW, N), jnp.float32),
            pltpu.SemaphoreType.DMA,
            pltpu.SemaphoreType.DMA,
        ],
        compiler_params=pltpu.CompilerParams(use_tc_tiling_on_sc=False),
    )
    def gather(proj_hbm, idx_hbm, out_hbm, idx_v, rows0, rows1, sem0, sem1):
        wid = lax.axis_index("s") * NC + lax.axis_index("c")
        base = wid * b_per_w
        pltpu.sync_copy(idx_hbm.at[pl.ds(wid * rows_per_w, rows_per_w)], idx_v)
        bufs = (rows0, rows1)
        sems = (sem0, sem1)

        def fire(g2, b):
            g = g2 * NB + b
            return [
                pltpu.async_copy(
                    proj_hbm.at[idx_v.at[g * G + r]],
                    bufs[b].at[pl.ds(r * IW, IW)],
                    sems[b],
                )
                for r in range(G)
            ]

        def drain(g2, b, copies):
            g = g2 * NB + b
            for c in copies:
                c.wait()
            pltpu.sync_copy(bufs[b], out_hbm.at[pl.ds(base + g * G * IW, G * IW)])

        def step(g2, carry):
            copies = [fire(g2, b) for b in range(NB)]
            for b in range(NB):
                drain(g2, b, copies[b])
            return carry

        lax.fori_loop(0, n_steps, step, 0)

    return gather


def kernel(text, emb_table, fc_w, fc_b):
    Bt, S = text.shape
    V, D = emb_table.shape
    N = fc_w.shape[0]
    NP = 8
    VP = 1048576

    PACK = 128 // NP
    VQ = VP // PACK

    wt = jnp.zeros((D, NP), jnp.float32).at[:, :N].set(fc_w.T)
    ar = jnp.arange(PACK)
    w512 = (
        jnp.zeros((PACK, D, PACK, NP), jnp.float32)
        .at[ar, :, ar, :]
        .set(jnp.broadcast_to(wt[None, :, :], (PACK, D, NP)))
        .reshape(D * PACK, PACK * NP)
    )
    b128 = jnp.tile(jnp.zeros((NP,), jnp.float32).at[:N].set(fc_b), PACK).reshape(1, 128)

    embt = jnp.pad(emb_table.T, ((0, 0), (0, VP - V)))
    proj = _project_t(embt, w512, b128, PACK).reshape(VP, NP)
    tix = text.astype(jnp.int32)
    tix = ((tix % VQ) * PACK) + (tix // VQ)
    idx = tix.reshape(-1, 128)
    out = _make_gather(VP, NP, idx.size)(proj, idx)
    return out[:, :N].reshape(Bt, S, N)


# speedup vs baseline: 4.4205x; 1.1562x over previous
import functools

import jax
import jax.numpy as jnp
from jax import lax
from jax.experimental import pallas as pl
from jax.experimental.pallas import tpu as pltpu
from jax.experimental.pallas import tpu_sc as plsc


def _make_proj_body(PACK):
    def body(*refs):
        x_refs = refs[:PACK]
        w_ref, b_ref, out_ref = refs[PACK:]
        x = jnp.concatenate([r[...] for r in x_refs], axis=0)
        out_ref[...] = (
            lax.dot_general(
                x, w_ref[...], (((0,), (0,)), ((), ())),
                preferred_element_type=jnp.float32,
            )
            + b_ref[...]
        )

    return body


def _project_t(embt, w512, b128, PACK):
    D, VP = embt.shape
    VQ = VP // PACK
    CB = 4096
    assert VQ % CB == 0
    grid = VQ // CB
    nb = VQ // CB

    def mk_index_map(p):
        return lambda i: (0, p * nb + i)

    return pl.pallas_call(
        _make_proj_body(PACK),
        grid=(grid,),
        in_specs=[
            pl.BlockSpec((D, CB), mk_index_map(p)) for p in range(PACK)
        ]
        + [
            pl.BlockSpec((D * PACK, 128), lambda i: (0, 0)),
            pl.BlockSpec((1, 128), lambda i: (0, 0)),
        ],
        out_specs=pl.BlockSpec((CB, 128), lambda i: (i, 0)),
        out_shape=jax.ShapeDtypeStruct((VQ, 128), jnp.float32),
    )(*([embt] * PACK), w512, b128)


@functools.lru_cache(maxsize=None)
def _make_gather(V, N, B):
    info = plsc.get_sparse_core_info()
    NC, NS = info.num_cores, info.num_subcores
    NW = NC * NS
    assert B % NW == 0
    b_per_w = B // NW
    mesh = plsc.VectorSubcoreMesh(core_axis_name="c", subcore_axis_name="s")

    IW = 128
    assert b_per_w % IW == 0
    rows_per_w = b_per_w // IW
    G = 10
    NB = 2
    assert rows_per_w % (G * NB) == 0
    n_steps = rows_per_w // (G * NB)

    @functools.partial(
        pl.kernel,
        mesh=mesh,
        out_type=jax.ShapeDtypeStruct((B, N), jnp.float32),
        scratch_types=[
            pltpu.VMEM((rows_per_w, IW), jnp.int32),
            pltpu.VMEM((G * IW, N), jnp.float32),
            pltpu.VMEM((G * IW, N), jnp.float32),
            pltpu.SemaphoreType.DMA,
            pltpu.SemaphoreType.DMA,
        ],
        compiler_params=pltpu.CompilerParams(use_tc_tiling_on_sc=False),
    )
    def gather(proj_hbm, idx_hbm, out_hbm, idx_v, rows0, rows1, sem0, sem1):
        wid = lax.axis_index("s") * NC + lax.axis_index("c")
        base = wid * b_per_w
        pltpu.sync_copy(idx_hbm.at[pl.ds(wid * rows_per_w, rows_per_w)], idx_v)
        bufs = (rows0, rows1)
        sems = (sem0, sem1)

        def fire(g2, b):
            g = g2 * NB + b
            return [
                pltpu.async_copy(
                    proj_hbm.at[idx_v.at[g * G + r]],
                    bufs[b].at[pl.ds(r * IW, IW)],
                    sems[b],
                )
                for r in range(G)
            ]

        def drain(g2, b, copies):
            g = g2 * NB + b
            for c in copies:
                c.wait()
            pltpu.sync_copy(bufs[b], out_hbm.at[pl.ds(base + g * G * IW, G * IW)])

        def step(g2, carry):
            copies = [fire(g2, b) for b in range(NB)]
            for b in range(NB):
                drain(g2, b, copies[b])
            return carry

        lax.fori_loop(0, n_steps, step, 0)

    return gather


def kernel(text, emb_table, fc_w, fc_b):
    Bt, S = text.shape
    V, D = emb_table.shape
    N = fc_w.shape[0]
    NP = 8
    VP = 1048576

    PACK = 128 // NP
    VQ = VP // PACK

    wt = jnp.zeros((D, NP), jnp.float32).at[:, :N].set(fc_w.T)
    ar = jnp.arange(PACK)
    w512 = (
        jnp.zeros((PACK, D, PACK, NP), jnp.float32)
        .at[ar, :, ar, :]
        .set(jnp.broadcast_to(wt[None, :, :], (PACK, D, NP)))
        .reshape(D * PACK, PACK * NP)
    )
    b128 = jnp.tile(jnp.zeros((NP,), jnp.float32).at[:N].set(fc_b), PACK).reshape(1, 128)

    embt = jnp.pad(emb_table.T, ((0, 0), (0, VP - V)))
    proj = _project_t(embt, w512, b128, PACK).reshape(VP, NP)
    tix = text.astype(jnp.int32).T
    tix = ((tix % VQ) * PACK) + (tix // VQ)
    idx = tix.reshape(-1, 128)
    out = _make_gather(VP, NP, idx.size)(proj, idx)
    return out.reshape(S, Bt, NP)[:, :, :N].transpose(1, 0, 2)
